# baseline (speedup 1.0000x reference)
import functools

import jax
import jax.numpy as jnp
from jax import lax
from jax.experimental import pallas as pl
from jax.experimental.pallas import tpu as pltpu
from jax.experimental.pallas import tpu_sc as plsc

_B, _N, _NF = 16, 256, 64
_R = _B * _N
_NW = 32
_RPW = _R // _NW
_L = 16


def _sc_norm_body(adj_hbm, norm_hbm, wmax_hbm, chunk, cnt, maxbuf):
    w = lax.axis_index("s") * 2 + lax.axis_index("c")
    base = w * _RPW
    pltpu.sync_copy(adj_hbm.at[pl.ds(base, _RPW)], chunk)
    lane = lax.iota(jnp.int32, _L)
    lmax = jnp.zeros((_L,), jnp.float32)
    for g in range(_RPW // _L):
        gv = jnp.zeros((_L,), jnp.float32)
        for r in range(_L):
            row = g * _L + r
            acc = chunk[row, pl.ds(0, _L)]
            for j in range(1, _N // _L):
                acc = acc + chunk[row, pl.ds(_L * j, _L)]
            s = acc[0]
            for l in range(1, _L):
                s = s + acc[l]
            gv = jnp.where(lane == r, jnp.maximum(s, 1.0), gv)
        cnt[pl.ds(g * _L, _L)] = gv
        lmax = jnp.maximum(lmax, gv)
    maxbuf[...] = lmax
    pltpu.sync_copy(cnt, norm_hbm.at[pl.ds(base, _RPW)])
    pltpu.sync_copy(maxbuf, wmax_hbm.at[pl.ds(w * _L, _L)])


_sc_norm = functools.partial(
    pl.kernel,
    out_type=(
        jax.ShapeDtypeStruct((_R,), jnp.float32),
        jax.ShapeDtypeStruct((_NW * _L,), jnp.float32),
    ),
    mesh=plsc.VectorSubcoreMesh(core_axis_name="c", subcore_axis_name="s"),
    scratch_types=[
        pltpu.VMEM((_RPW, _N), jnp.float32),
        pltpu.VMEM((_RPW,), jnp.float32),
        pltpu.VMEM((_L,), jnp.float32),
    ],
)(_sc_norm_body)


_G = 4


def _mpnn_kernel(state_ref, adj_ref, rn_ref, ng_ref, wi_ref, we_ref,
                 wmsg_ref, wupd_ref, wpool_ref, wread_ref, out_ref):
    for g in range(_G):
        a = adj_ref[g]
        s = state_ref[g, 0]
        rn = rn_ref[g, 0]

        e = jnp.maximum(we_ref[0][None, :] + s[:, None] * we_ref[1][None, :],
                        0.0)
        sedge = lax.dot_general(a, e, (((0,), (0,)), ((), ())),
                                preferred_element_type=jnp.float32, precision=lax.Precision.HIGHEST)
        sedge = sedge * rn[:, None]
        col = lax.broadcasted_iota(jnp.int32, (_N, _NF), 1)
        ee = jnp.where(col == _NF - 1, ng_ref[g, 0][:, None], sedge)
        ee = jnp.maximum(ee, 0.0)
        cur = jnp.maximum(s[:, None] * wi_ref[0][None, :], 0.0)
        for i in range(3):
            agg = lax.dot_general(a, cur, (((0,), (0,)), ((), ())),
                                  preferred_element_type=jnp.float32, precision=lax.Precision.HIGHEST)
            agg = agg * rn[:, None]
            msg = jnp.maximum(
                jnp.dot(agg, wmsg_ref[i, :_NF, :],
                        preferred_element_type=jnp.float32, precision=lax.Precision.HIGHEST)
                + jnp.dot(ee, wmsg_ref[i, _NF:, :],
                          preferred_element_type=jnp.float32, precision=lax.Precision.HIGHEST),
                0.0)
            cur = jnp.maximum(
                jnp.dot(cur, wupd_ref[i, :_NF, :],
                        preferred_element_type=jnp.float32, precision=lax.Precision.HIGHEST)
                + jnp.dot(msg, wupd_ref[i, _NF:, :],
                          preferred_element_type=jnp.float32, precision=lax.Precision.HIGHEST),
                0.0)

        hp = jnp.dot((jnp.sum(cur, axis=0) / _N)[None, :], wpool_ref[...],
                     preferred_element_type=jnp.float32, precision=lax.Precision.HIGHEST)
        c0 = jnp.sum(jnp.maximum(hp[0], 0.0) * wread_ref[0, :_NF])
        out_ref[g, 0] = c0 + jnp.sum(cur * wread_ref[0, _NF:][None, :], axis=1)


def _impl(embeddings, state, adj, W_init, W_edge, W_msg, W_upd, W_pool,
          W_read, b_read):
    del embeddings
    norm_flat, wmax = _sc_norm(adj.reshape(_R, _N))
    rn = jnp.reciprocal(norm_flat)
    ng = norm_flat / jnp.max(wmax)

    we_pad = jnp.pad(W_edge, ((0, 0), (0, 1)))
    wread = W_read.reshape(1, 2 * _NF)

    full = lambda *shape: pl.BlockSpec(shape, lambda b: (0,) * len(shape))
    out = pl.pallas_call(
        _mpnn_kernel,
        grid=(_B // _G,),
        in_specs=[
            pl.BlockSpec((_G, 1, _N), lambda b: (b, 0, 0)),
            pl.BlockSpec((_G, _N, _N), lambda b: (b, 0, 0)),
            pl.BlockSpec((_G, 1, _N), lambda b: (b, 0, 0)),
            pl.BlockSpec((_G, 1, _N), lambda b: (b, 0, 0)),
            full(1, _NF),
            full(2, _NF),
            full(3, 2 * _NF, _NF),
            full(3, 2 * _NF, _NF),
            full(_NF, _NF),
            full(1, 2 * _NF),
        ],
        out_specs=pl.BlockSpec((_G, 1, _N), lambda b: (b, 0, 0)),
        out_shape=jax.ShapeDtypeStruct((_B, 1, _N), jnp.float32),
        compiler_params=pltpu.CompilerParams(
            dimension_semantics=("parallel",)),
    )(state.reshape(_B, 1, _N), adj, rn.reshape(_B, 1, _N),
      ng.reshape(_B, 1, _N), W_init, we_pad, W_msg, W_upd, W_pool, wread)
    return out.reshape(_B, _N) + b_read[0]


kernel = jax.jit(_impl)
